# Initial kernel scaffold; baseline (speedup 1.0000x reference)
import jax
import jax.numpy as jnp
from jax.experimental import pallas as pl


def kernel(hidden_states, router_w, gate_w, up_w, down_w):
    raise NotImplementedError("write your pallas kernel here")


# speedup vs baseline: 1.5499x; 1.5499x over previous
import functools

import jax
import jax.numpy as jnp
from jax.experimental import pallas as pl
from jax.experimental.pallas import tpu as pltpu

HIDDEN = 1024
FFN = 512
NUM_EXPERTS = 8
TOP_K = 2

TBLK = 256


def _moe_body(x_ref, rw_ref, g_ref, u_ref, d_ref, out_ref, comb_s):
    e = pl.program_id(1)

    @pl.when(e == 0)
    def _router():
        x = x_ref[...]
        logits = jnp.dot(x, rw_ref[...], preferred_element_type=jnp.float32)
        probs = jax.nn.softmax(logits, axis=-1)
        lane = jax.lax.broadcasted_iota(jnp.int32, probs.shape, 1)
        m1 = jnp.max(probs, axis=-1, keepdims=True)
        i1 = jnp.min(jnp.where(probs == m1, lane, NUM_EXPERTS), axis=-1,
                     keepdims=True)
        oh1 = lane == i1
        masked = jnp.where(oh1, -jnp.inf, probs)
        m2 = jnp.max(masked, axis=-1, keepdims=True)
        i2 = jnp.min(jnp.where(masked == m2, lane, NUM_EXPERTS), axis=-1,
                     keepdims=True)
        oh2 = lane == i2
        s = m1 + m2
        comb_s[...] = jnp.where(oh1, m1 / s, 0.0) + jnp.where(oh2, m2 / s, 0.0)

    xb = x_ref[...].astype(jnp.bfloat16)
    a = jnp.dot(xb, g_ref[0], preferred_element_type=jnp.float32)
    b = jnp.dot(xb, u_ref[0], preferred_element_type=jnp.float32)
    h = (jax.nn.silu(a) * b).astype(jnp.bfloat16)
    y = jnp.dot(h, d_ref[0], preferred_element_type=jnp.float32)
    oh_e = (jax.lax.broadcasted_iota(jnp.int32, (NUM_EXPERTS, 1), 0) == e
            ).astype(jnp.float32)
    w = jnp.dot(comb_s[...], oh_e, preferred_element_type=jnp.float32)
    z = w * y

    @pl.when(e == 0)
    def _init():
        out_ref[...] = z

    @pl.when(e != 0)
    def _acc():
        out_ref[...] += z


@functools.partial(jax.jit, static_argnames=())
def _moe(x, rw, gw, uw, dw):
    T = x.shape[0]
    grid = (T // TBLK, NUM_EXPERTS)
    return pl.pallas_call(
        _moe_body,
        grid=grid,
        in_specs=[
            pl.BlockSpec((TBLK, HIDDEN), lambda t, e: (t, 0)),
            pl.BlockSpec((HIDDEN, NUM_EXPERTS), lambda t, e: (0, 0)),
            pl.BlockSpec((1, HIDDEN, FFN), lambda t, e: (e, 0, 0)),
            pl.BlockSpec((1, HIDDEN, FFN), lambda t, e: (e, 0, 0)),
            pl.BlockSpec((1, FFN, HIDDEN), lambda t, e: (e, 0, 0)),
        ],
        out_specs=pl.BlockSpec((TBLK, HIDDEN), lambda t, e: (t, 0)),
        out_shape=jax.ShapeDtypeStruct((T, HIDDEN), jnp.float32),
        scratch_shapes=[pltpu.VMEM((TBLK, NUM_EXPERTS), jnp.float32)],
    )(x, rw, gw, uw, dw)


def kernel(hidden_states, router_w, gate_w, up_w, down_w):
    B, S, H = hidden_states.shape
    x = hidden_states.reshape(-1, H)
    out = _moe(x, router_w,
               gate_w.astype(jnp.bfloat16),
               up_w.astype(jnp.bfloat16),
               down_w.astype(jnp.bfloat16))
    return out.reshape(B, S, H)
